# baseline (speedup 1.0000x reference)
import functools

import jax
import jax.numpy as jnp
from jax import lax
from jax.experimental import pallas as pl
from jax.experimental.pallas import tpu as pltpu
from jax.experimental.pallas import tpu_sc as plsc

NC, NS = 2, 16
NW = NC * NS
CHUNK = 128
NB = 4
L = 16


@functools.lru_cache(maxsize=None)
def _build_gather(nb_batch, g, two, d):
    n_rows = nb_batch * g * two
    per_w = n_rows // NW
    n_chunks = per_w // CHUNK
    bpw = nb_batch // NW
    pair = g * two
    mesh = plsc.VectorSubcoreMesh(core_axis_name="c", subcore_axis_name="s")

    @functools.partial(
        pl.kernel,
        mesh=mesh,
        out_type=jax.ShapeDtypeStruct((n_rows, d), jnp.float32),
        scratch_types=[
            pltpu.VMEM((bpw, g, two), jnp.int32),
            pltpu.VMEM((n_chunks, CHUNK), jnp.int32),
            pltpu.VMEM((NB, CHUNK, d), jnp.float32),
            pltpu.SemaphoreType.DMA((NB,)),
            pltpu.SemaphoreType.DMA((NB,)),
        ],
        compiler_params=pltpu.CompilerParams(
            use_tc_tiling_on_sc=False, needs_layout_passes=False),
    )
    def gather_kernel(table_hbm, x_hbm, out_hbm, xv, idx_v, bufs, gsem, wsem):
        wid = lax.axis_index("c") * NS + lax.axis_index("s")
        pltpu.sync_copy(x_hbm.at[pl.ds(wid * bpw, bpw)], xv)

        def unpack(j, carry):
            lanes = lax.iota(jnp.int32, L)
            for s in range(CHUNK // L):
                flat = j * CHUNK + s * L + lanes
                bi = lax.div(flat, jnp.int32(pair))
                r = lax.rem(flat, jnp.int32(pair))
                v = plsc.load_gather(
                    xv, [bi, lax.div(r, jnp.int32(two)),
                         lax.rem(r, jnp.int32(two))])
                idx_v[j, pl.ds(s * L, L)] = v
            return carry

        lax.fori_loop(0, n_chunks, unpack, 0)

        base = wid * per_w

        def gather_start(i, b):
            pltpu.async_copy(table_hbm.at[idx_v.at[i]], bufs.at[b], gsem.at[b])

        def gather_wait(i, b):
            pltpu.make_async_copy(
                table_hbm.at[idx_v.at[i]], bufs.at[b], gsem.at[b]).wait()

        def write_start(i, b):
            pltpu.async_copy(
                bufs.at[b], out_hbm.at[pl.ds(base + i * CHUNK, CHUNK)],
                wsem.at[b])

        def write_wait(i, b):
            pltpu.make_async_copy(
                bufs.at[b], out_hbm.at[pl.ds(base + i * CHUNK, CHUNK)],
                wsem.at[b]).wait()

        for b in range(NB):
            gather_start(b, b)

        def group(gi, carry):
            for b in range(NB):
                i = gi * NB + b
                gather_wait(i, b)
                write_start(i, b)
                write_wait(i, b)
                gather_start(i + NB, b)
            return carry

        lax.fori_loop(0, n_chunks // NB - 1, group, 0)

        for b in range(NB):
            i = n_chunks - NB + b
            gather_wait(i, b)
            write_start(i, b)
            write_wait(i, b)

    return gather_kernel


def kernel(x, table):
    b, g, two = x.shape
    d = table.shape[1]
    rows = _build_gather(b, g, two, d)(table, x)
    return rows.reshape(b, g, two * d)


# speedup vs baseline: 1.4095x; 1.4095x over previous
import functools

import jax
import jax.numpy as jnp
from jax import lax
from jax.experimental import pallas as pl
from jax.experimental.pallas import tpu as pltpu
from jax.experimental.pallas import tpu_sc as plsc

NC, NS = 2, 16
NW = NC * NS
NB = 4
L = 16


@functools.lru_cache(maxsize=None)
def _build_gather(nb_batch, g, two, d):
    pair = g * two
    bpw = nb_batch // NW
    mesh = plsc.VectorSubcoreMesh(core_axis_name="c", subcore_axis_name="s")

    @functools.partial(
        pl.kernel,
        mesh=mesh,
        out_type=jax.ShapeDtypeStruct((nb_batch, g, two * d), jnp.float32),
        scratch_types=[
            pltpu.VMEM((bpw, pair), jnp.int32),
            pltpu.VMEM((NB, pair, d), jnp.float32),
            pltpu.VMEM((NB, g, two * d), jnp.float32),
            pltpu.SemaphoreType.DMA((NB,)),
            pltpu.SemaphoreType.DMA((NB,)),
        ],
        compiler_params=pltpu.CompilerParams(use_tc_tiling_on_sc=False),
    )
    def gather_kernel(table_hbm, idx_hbm, out_hbm, idx_v, bufs, wbufs,
                      gsem, wsem):
        wid = lax.axis_index("c") * NS + lax.axis_index("s")
        pltpu.sync_copy(idx_hbm.at[wid], idx_v)
        base = wid * bpw

        def gather_start(i, b):
            pltpu.async_copy(table_hbm.at[idx_v.at[i]], bufs.at[b],
                             gsem.at[b])

        def gather_wait(i, b):
            pltpu.make_async_copy(table_hbm.at[idx_v.at[i]], bufs.at[b],
                                  gsem.at[b]).wait()

        def repack(b):
            for r in range(pair):
                for s in range(d // L):
                    f = r * d + s * L
                    wbufs[b, f // (two * d), pl.ds(f % (two * d), L)] = (
                        bufs[b, r, pl.ds(s * L, L)])

        def write_start(i, b):
            pltpu.async_copy(wbufs.at[b], out_hbm.at[base + i], wsem.at[b])

        def write_wait(i, b):
            pltpu.make_async_copy(wbufs.at[b], out_hbm.at[base + i],
                                  wsem.at[b]).wait()

        for b in range(NB):
            gather_start(b, b)

        def group(gi, carry):
            for b in range(NB):
                i = gi * NB + b
                gather_wait(i, b)
                repack(b)
                write_start(i, b)
                write_wait(i, b)
                gather_start(i + NB, b)
            return carry

        lax.fori_loop(0, bpw // NB - 1, group, 0)

        for b in range(NB):
            i = bpw - NB + b
            gather_wait(i, b)
            repack(b)
            write_start(i, b)
            write_wait(i, b)

    return gather_kernel


def kernel(x, table):
    b, g, two = x.shape
    d = table.shape[1]
    idx = x.reshape(NW, b // NW, g * two)
    return _build_gather(b, g, two, d)(table, idx)


# speedup vs baseline: 1.9320x; 1.3707x over previous
import functools

import jax
import jax.numpy as jnp
from jax import lax
from jax.experimental import pallas as pl
from jax.experimental.pallas import tpu as pltpu
from jax.experimental.pallas import tpu_sc as plsc

NC, NS = 2, 16
NW = NC * NS
NB = 4
L = 16


@functools.lru_cache(maxsize=None)
def _build_gather(nb_batch, g, two, d):
    pair = g * two
    bpw = nb_batch // NW
    mesh = plsc.VectorSubcoreMesh(core_axis_name="c", subcore_axis_name="s")

    @functools.partial(
        pl.kernel,
        mesh=mesh,
        out_type=jax.ShapeDtypeStruct((nb_batch, g + 6, two * d),
                                      jnp.float32),
        scratch_types=[
            pltpu.VMEM((bpw, pair), jnp.int32),
            pltpu.VMEM((NB, pair, d), jnp.float32),
            pltpu.VMEM((NB, g, two * d), jnp.float32),
            pltpu.SemaphoreType.DMA((NB,)),
            pltpu.SemaphoreType.DMA((NB,)),
        ],
        compiler_params=pltpu.CompilerParams(use_tc_tiling_on_sc=False),
    )
    def gather_kernel(table_hbm, idx_hbm, out_hbm, idx_v, bufs, wbufs,
                      gsem, wsem):
        wid = lax.axis_index("c") * NS + lax.axis_index("s")
        pltpu.sync_copy(idx_hbm.at[wid], idx_v)
        base = wid * bpw

        def gather_start(i, b):
            pltpu.async_copy(table_hbm.at[idx_v.at[i]], bufs.at[b],
                             gsem.at[b])

        def gather_wait(i, b):
            pltpu.make_async_copy(table_hbm.at[idx_v.at[i]], bufs.at[b],
                                  gsem.at[b]).wait()

        def repack(b):
            for r in range(pair):
                for s in range(d // L):
                    f = r * d + s * L
                    wbufs[b, f // (two * d), pl.ds(f % (two * d), L)] = (
                        bufs[b, r, pl.ds(s * L, L)])

        def write_start(i, b):
            pltpu.async_copy(wbufs.at[b],
                             out_hbm.at[base + i].at[pl.ds(0, g)],
                             wsem.at[b])

        def write_wait(i, b):
            pltpu.make_async_copy(wbufs.at[b],
                                  out_hbm.at[base + i].at[pl.ds(0, g)],
                                  wsem.at[b]).wait()

        for b in range(NB):
            gather_start(b, b)

        def group(gi, carry):
            for b in range(NB):
                i = gi * NB + b
                gather_wait(i, b)
                repack(b)
                write_start(i, b)
                write_wait(i, b)
                gather_start(i + NB, b)
            return carry

        lax.fori_loop(0, bpw // NB - 1, group, 0)

        for b in range(NB):
            i = bpw - NB + b
            gather_wait(i, b)
            repack(b)
            write_start(i, b)
            write_wait(i, b)

    return gather_kernel


def kernel(x, table):
    b, g, two = x.shape
    d = table.shape[1]
    idx = x.reshape(NW, b // NW, g * two)
    padded = _build_gather(b, g, two, d)(table, idx)
    return padded[:, :g, :]


# speedup vs baseline: 2.0887x; 1.0811x over previous
import functools

import jax
import jax.numpy as jnp
from jax import lax
from jax.experimental import pallas as pl
from jax.experimental.pallas import tpu as pltpu
from jax.experimental.pallas import tpu_sc as plsc

NC, NS = 2, 16
NW = NC * NS
NB = 8
L = 16


@functools.lru_cache(maxsize=None)
def _build_gather(nb_batch, g, two, d):
    pair = g * two
    bpw = nb_batch // NW
    mesh = plsc.VectorSubcoreMesh(core_axis_name="c", subcore_axis_name="s")

    @functools.partial(
        pl.kernel,
        mesh=mesh,
        out_type=jax.ShapeDtypeStruct((nb_batch, g + 6, two * d),
                                      jnp.float32),
        scratch_types=[
            pltpu.VMEM((bpw, pair), jnp.int32),
            pltpu.VMEM((NB, pair, d), jnp.float32),
            pltpu.VMEM((NB, g, two * d), jnp.float32),
            pltpu.SemaphoreType.DMA((NB,)),
            pltpu.SemaphoreType.DMA((NB,)),
        ],
        compiler_params=pltpu.CompilerParams(use_tc_tiling_on_sc=False),
    )
    def gather_kernel(table_hbm, idx_hbm, out_hbm, idx_v, bufs, wbufs,
                      gsem, wsem):
        wid = lax.axis_index("c") * NS + lax.axis_index("s")
        pltpu.sync_copy(idx_hbm.at[wid], idx_v)
        base = wid * bpw

        def gather_start(i, b):
            pltpu.async_copy(table_hbm.at[idx_v.at[i]], bufs.at[b],
                             gsem.at[b])

        def gather_wait(i, b):
            pltpu.make_async_copy(table_hbm.at[idx_v.at[i]], bufs.at[b],
                                  gsem.at[b]).wait()

        RU = 20
        RW = RU * d // (two * d)

        def repack(b):
            def rbody(k, carry):
                for u in range(RU):
                    for s in range(d // L):
                        fo = u * d + s * L
                        wbufs[b, k * RW + fo // (two * d),
                              pl.ds(fo % (two * d), L)] = (
                            bufs[b, k * RU + u, pl.ds(s * L, L)])
                return carry

            lax.fori_loop(0, pair // RU, rbody, 0)

        def write_start(i, b):
            pltpu.async_copy(wbufs.at[b],
                             out_hbm.at[base + i].at[pl.ds(0, g)],
                             wsem.at[b])

        def write_wait(i, b):
            pltpu.make_async_copy(wbufs.at[b],
                                  out_hbm.at[base + i].at[pl.ds(0, g)],
                                  wsem.at[b]).wait()

        for b in range(NB):
            gather_start(b, b)

        for b in range(NB):
            gather_wait(b, b)
            repack(b)
            write_start(b, b)
            gather_start(b + NB, b)

        def group(gi, carry):
            for b in range(NB):
                i = gi * NB + b
                gather_wait(i, b)
                write_wait(i - NB, b)
                repack(b)
                write_start(i, b)
                gather_start(i + NB, b)
            return carry

        lax.fori_loop(1, bpw // NB - 1, group, 0)

        for b in range(NB):
            i = bpw - NB + b
            gather_wait(i, b)
            write_wait(i - NB, b)
            repack(b)
            write_start(i, b)

        for b in range(NB):
            write_wait(bpw - NB + b, b)

    return gather_kernel


def kernel(x, table):
    b, g, two = x.shape
    d = table.shape[1]
    idx = x.reshape(NW, b // NW, g * two)
    padded = _build_gather(b, g, two, d)(table, idx)
    return padded[:, :g, :]


# speedup vs baseline: 2.1736x; 1.0407x over previous
import functools

import jax
import jax.numpy as jnp
from jax import lax
from jax.experimental import pallas as pl
from jax.experimental.pallas import tpu as pltpu
from jax.experimental.pallas import tpu_sc as plsc

NC, NS = 2, 16
NW = NC * NS
NB = 8
L = 16


@functools.lru_cache(maxsize=None)
def _build_gather(nb_batch, g, two, d):
    pair = g * two
    bpw = nb_batch // NW
    mesh = plsc.VectorSubcoreMesh(core_axis_name="c", subcore_axis_name="s")

    @functools.partial(
        pl.kernel,
        mesh=mesh,
        out_type=jax.ShapeDtypeStruct((nb_batch, g + 6, two * d),
                                      jnp.float32),
        scratch_types=[
            pltpu.VMEM((bpw, pair), jnp.int32),
            pltpu.VMEM((NB, two, g), jnp.int32),
            pltpu.VMEM((NB, two, g, d), jnp.float32),
            pltpu.SemaphoreType.DMA((NB,)),
            pltpu.SemaphoreType.DMA((NB,)),
        ],
        compiler_params=pltpu.CompilerParams(
            use_tc_tiling_on_sc=False, needs_layout_passes=False),
    )
    def gather_kernel(table_hbm, idx_hbm, out_hbm, idx_v, idxeo, bufs,
                      gsem, wsem):
        wid = lax.axis_index("c") * NS + lax.axis_index("s")
        pltpu.sync_copy(idx_hbm.at[wid], idx_v)
        base = wid * bpw

        def build_lists(i, b):
            lanes = lax.iota(jnp.int32, L)
            bi = i + lanes * 0
            for k in range(two):
                for off in (0, 16, 32, 34):
                    ji = (off + lanes) * two + k
                    idxeo[b, k, pl.ds(off, L)] = plsc.load_gather(
                        idx_v, [bi, ji])

        def gather_start(b):
            for k in range(two):
                pltpu.async_copy(table_hbm.at[idxeo.at[b, k]],
                                 bufs.at[b, k], gsem.at[b])

        def gather_wait(b):
            for k in range(two):
                pltpu.make_async_copy(table_hbm.at[idxeo.at[b, k]],
                                      bufs.at[b, k], gsem.at[b]).wait()

        def write_start(i, b):
            for k in range(two):
                pltpu.async_copy(
                    bufs.at[b, k],
                    out_hbm.at[base + i].at[pl.ds(0, g), pl.ds(k * d, d)],
                    wsem.at[b])

        def write_wait(i, b):
            for k in range(two):
                pltpu.make_async_copy(
                    bufs.at[b, k],
                    out_hbm.at[base + i].at[pl.ds(0, g), pl.ds(k * d, d)],
                    wsem.at[b]).wait()

        for b in range(NB):
            build_lists(b, b)
            gather_start(b)

        def group(gi, carry):
            for b in range(NB):
                i = gi * NB + b
                gather_wait(b)
                build_lists(i + NB, b)
                write_start(i, b)
                write_wait(i, b)
                gather_start(b)
            return carry

        lax.fori_loop(0, bpw // NB - 1, group, 0)

        for b in range(NB):
            i = bpw - NB + b
            gather_wait(b)
            write_start(i, b)
            write_wait(i, b)

    return gather_kernel


def kernel(x, table):
    b, g, two = x.shape
    d = table.shape[1]
    idx = x.reshape(NW, b // NW, g * two)
    padded = _build_gather(b, g, two, d)(table, idx)
    return padded[:, :g, :]
